# baseline (speedup 1.0000x reference)
import jax
import jax.numpy as jnp
from jax.experimental import pallas as pl
from jax.experimental.pallas import tpu as pltpu

_T = 256
_R = 4


def _mm_kernel(*refs):
    x_ref = refs[0]
    nws = refs[1:1 + _R]
    ows = refs[1 + _R:1 + 2 * _R]
    o1_ref, o2_ref = refs[1 + 2 * _R], refs[2 + 2 * _R]
    x = x_ref[...]
    rt = _T // _R
    dn = (((1,), (1,)), ((), ()))
    for i in range(_R):
        o1_ref[:, i * rt:(i + 1) * rt] = jax.lax.dot_general(
            x, nws[i][...], dn, preferred_element_type=jnp.float32)
        o2_ref[:, i * rt:(i + 1) * rt] = jax.lax.dot_general(
            x, ows[i][...], dn, preferred_element_type=jnp.float32)


def kernel(x, new_weight, orig_weight):
    M, K = x.shape
    N = new_weight.shape[0]
    rt = _T // _R
    wspec = [pl.BlockSpec((rt, K), (lambda i: (lambda j: (j * _R + i, 0)))(i))
             for i in range(_R)]
    out1, out2 = pl.pallas_call(
        _mm_kernel,
        grid=(N // _T,),
        in_specs=[pl.BlockSpec((M, K), lambda j: (0, 0))] + wspec + wspec,
        out_specs=[
            pl.BlockSpec((M, _T), lambda j: (0, j)),
            pl.BlockSpec((M, _T), lambda j: (0, j)),
        ],
        out_shape=[
            jax.ShapeDtypeStruct((M, N), jnp.float32),
            jax.ShapeDtypeStruct((M, N), jnp.float32),
        ],
        compiler_params=pltpu.CompilerParams(
            dimension_semantics=("arbitrary",)),
    )(x, *([new_weight] * _R), *([orig_weight] * _R))
    return (out1, out2)


# speedup vs baseline: 1.0401x; 1.0401x over previous
import jax
import jax.numpy as jnp
from jax.experimental import pallas as pl
from jax.experimental.pallas import tpu as pltpu

_T = 256
_R = 2
_KS = 4


def _mm_kernel(*refs):
    ns = _R * _KS
    x_ref = refs[0]
    nws = refs[1:1 + ns]
    ows = refs[1 + ns:1 + 2 * ns]
    o1_ref, o2_ref = refs[1 + 2 * ns], refs[2 + 2 * ns]
    x = x_ref[...]
    rt = _T // _R
    kq = x.shape[1] // _KS
    xs = [x[:, k * kq:(k + 1) * kq] for k in range(_KS)]
    dn = (((1,), (1,)), ((), ()))
    for r in range(_R):
        o1_ref[:, r * rt:(r + 1) * rt] = sum(
            jax.lax.dot_general(xs[k], nws[r * _KS + k][...], dn,
                                preferred_element_type=jnp.float32)
            for k in range(_KS))
        o2_ref[:, r * rt:(r + 1) * rt] = sum(
            jax.lax.dot_general(xs[k], ows[r * _KS + k][...], dn,
                                preferred_element_type=jnp.float32)
            for k in range(_KS))


def kernel(x, new_weight, orig_weight):
    M, K = x.shape
    N = new_weight.shape[0]
    rt = _T // _R
    wspec = [pl.BlockSpec(
        (rt, K // _KS),
        (lambda r, k: (lambda j: (j * _R + r, k)))(r, k))
        for r in range(_R) for k in range(_KS)]
    out1, out2 = pl.pallas_call(
        _mm_kernel,
        grid=(N // _T,),
        in_specs=[pl.BlockSpec((M, K), lambda j: (0, 0))] + wspec + wspec,
        out_specs=[
            pl.BlockSpec((M, _T), lambda j: (0, j)),
            pl.BlockSpec((M, _T), lambda j: (0, j)),
        ],
        out_shape=[
            jax.ShapeDtypeStruct((M, N), jnp.float32),
            jax.ShapeDtypeStruct((M, N), jnp.float32),
        ],
        compiler_params=pltpu.CompilerParams(
            dimension_semantics=("arbitrary",)),
    )(x, *([new_weight] * (_R * _KS)), *([orig_weight] * (_R * _KS)))
    return (out1, out2)
